# baseline (speedup 1.0000x reference)
import functools

import jax
import jax.numpy as jnp
from jax import lax
from jax.experimental import pallas as pl
from jax.experimental.pallas import tpu as pltpu
from jax.experimental.pallas import tpu_sc as plsc

N_TOKENS = 1000002
EMB_DIM = 64
BATCH = 4096
HIST_LEN = 200

_INFO = plsc.get_sparse_core_info()
_NC = _INFO.num_cores
_NS = _INFO.num_subcores
_NW = _NC * _NS

_N = BATCH * HIST_LEN
_B_PER_W = _N // _NW
_CHUNK = 512
_N_CHUNKS = _B_PER_W // _CHUNK


@functools.partial(jax.jit, static_argnames=())
def _gather_sc(x_flat, table):
    mesh = plsc.VectorSubcoreMesh(core_axis_name="c", subcore_axis_name="s")

    @functools.partial(
        pl.kernel,
        mesh=mesh,
        out_type=jax.ShapeDtypeStruct((_N, EMB_DIM), jnp.float32),
        scratch_types=[
            pltpu.VMEM((_CHUNK,), jnp.int32),
            pltpu.VMEM((_CHUNK, EMB_DIM), jnp.float32),
            pltpu.SemaphoreType.DMA,
        ],
        compiler_params=pltpu.CompilerParams(use_tc_tiling_on_sc=False),
    )
    def k(x_hbm, table_hbm, out_hbm, idx_v, rows_v, sem):
        wid = lax.axis_index("s") * _NC + lax.axis_index("c")
        base = wid * _B_PER_W

        def body(i, carry):
            start = base + i * _CHUNK
            pltpu.sync_copy(x_hbm.at[pl.ds(start, _CHUNK)], idx_v)
            pltpu.async_copy(table_hbm.at[idx_v], rows_v, sem).wait()
            pltpu.sync_copy(rows_v, out_hbm.at[pl.ds(start, _CHUNK), :])
            return carry

        lax.fori_loop(0, _N_CHUNKS, body, 0)

    return k(x_flat, table)


def kernel(x, embedding_weight):
    x_flat = x.reshape(-1).astype(jnp.int32)
    out = _gather_sc(x_flat, embedding_weight)
    return out.reshape(BATCH, HIST_LEN, EMB_DIM)


# speedup vs baseline: 1.0437x; 1.0437x over previous
import functools

import jax
import jax.numpy as jnp
from jax import lax
from jax.experimental import pallas as pl
from jax.experimental.pallas import tpu as pltpu
from jax.experimental.pallas import tpu_sc as plsc

N_TOKENS = 1000002
EMB_DIM = 64
BATCH = 4096
HIST_LEN = 200

_INFO = plsc.get_sparse_core_info()
_NC = _INFO.num_cores
_NS = _INFO.num_subcores
_NW = _NC * _NS

_N = BATCH * HIST_LEN
_B_PER_W = _N // _NW
_CHUNK = 256
_NBUF = 4
_N_CHUNKS = _B_PER_W // _CHUNK
_N_GROUPS = _N_CHUNKS // _NBUF


@jax.jit
def _gather_sc(x_flat, table):
    mesh = plsc.VectorSubcoreMesh(core_axis_name="c", subcore_axis_name="s")

    @functools.partial(
        pl.kernel,
        mesh=mesh,
        out_type=jax.ShapeDtypeStruct((_N, EMB_DIM), jnp.float32),
        scratch_types=[
            pltpu.VMEM((_N_CHUNKS, _CHUNK), jnp.int32),
            pltpu.VMEM((_NBUF, _CHUNK, EMB_DIM), jnp.float32),
            pltpu.SemaphoreType.DMA((_NBUF,)),
            pltpu.SemaphoreType.DMA((_NBUF,)),
        ],
        compiler_params=pltpu.CompilerParams(use_tc_tiling_on_sc=False),
    )
    def k(x_hbm, table_hbm, out_hbm, idx_v, rows_v, gsem, ssem):
        wid = lax.axis_index("s") * _NC + lax.axis_index("c")
        base = wid * _B_PER_W
        pltpu.sync_copy(
            x_hbm.at[pl.ds(wid * _N_CHUNKS, _N_CHUNKS), :], idx_v
        )

        def group(g, carry):
            for b in range(_NBUF):
                i = g * _NBUF + b

                @pl.when(g > 0)
                def _wait_store():
                    pltpu.make_async_copy(
                        rows_v.at[b],
                        out_hbm.at[pl.ds(base, _CHUNK), :],
                        ssem.at[b],
                    ).wait()

                pltpu.make_async_copy(
                    table_hbm.at[idx_v.at[i]], rows_v.at[b], gsem.at[b]
                ).start()
            for b in range(_NBUF):
                i = g * _NBUF + b
                start = base + i * _CHUNK
                pltpu.make_async_copy(
                    table_hbm.at[idx_v.at[i]], rows_v.at[b], gsem.at[b]
                ).wait()
                pltpu.make_async_copy(
                    rows_v.at[b],
                    out_hbm.at[pl.ds(start, _CHUNK), :],
                    ssem.at[b],
                ).start()
            return carry

        lax.fori_loop(0, _N_GROUPS, group, 0)
        for b in range(_NBUF):
            pltpu.make_async_copy(
                rows_v.at[b],
                out_hbm.at[pl.ds(base, _CHUNK), :],
                ssem.at[b],
            ).wait()

    return k(x_flat, table)


def kernel(x, embedding_weight):
    x_2d = x.reshape(_N // _CHUNK, _CHUNK).astype(jnp.int32)
    out = _gather_sc(x_2d, embedding_weight)
    return out.reshape(BATCH, HIST_LEN, EMB_DIM)


# speedup vs baseline: 1.2635x; 1.2106x over previous
import functools

import jax
import jax.numpy as jnp
from jax import lax
from jax.experimental import pallas as pl
from jax.experimental.pallas import tpu as pltpu
from jax.experimental.pallas import tpu_sc as plsc

N_TOKENS = 1000002
EMB_DIM = 64
PAD_DIM = 128
BATCH = 4096
HIST_LEN = 200

_INFO = plsc.get_sparse_core_info()
_NC = _INFO.num_cores
_NS = _INFO.num_subcores
_NW = _NC * _NS

_N = BATCH * HIST_LEN
_B_PER_W = _N // _NW
_CHUNK = 128
_NBUF = 4
_N_CHUNKS = _B_PER_W // _CHUNK
_N_GROUPS = _N_CHUNKS // _NBUF


@jax.jit
def _gather_sc(x_2d, table_p):
    mesh = plsc.VectorSubcoreMesh(core_axis_name="c", subcore_axis_name="s")

    @functools.partial(
        pl.kernel,
        mesh=mesh,
        out_type=jax.ShapeDtypeStruct((_N, PAD_DIM), jnp.float32),
        scratch_types=[
            pltpu.VMEM((_N_CHUNKS, _CHUNK), jnp.int32),
            pltpu.VMEM((_NBUF, _CHUNK, PAD_DIM), jnp.float32),
            pltpu.SemaphoreType.DMA((_NBUF,)),
            pltpu.SemaphoreType.DMA((_NBUF,)),
        ],
        compiler_params=pltpu.CompilerParams(use_tc_tiling_on_sc=False),
    )
    def k(x_hbm, table_hbm, out_hbm, idx_v, rows_v, gsem, ssem):
        wid = lax.axis_index("s") * _NC + lax.axis_index("c")
        base = wid * _B_PER_W
        pltpu.sync_copy(
            x_hbm.at[pl.ds(wid * _N_CHUNKS, _N_CHUNKS), :], idx_v
        )

        def group(g, carry):
            for b in range(_NBUF):
                i = g * _NBUF + b

                @pl.when(g > 0)
                def _wait_store():
                    pltpu.make_async_copy(
                        rows_v.at[b, slice(None), pl.ds(0, EMB_DIM)],
                        out_hbm.at[pl.ds(base, _CHUNK), pl.ds(0, EMB_DIM)],
                        ssem.at[b],
                    ).wait()

                pltpu.make_async_copy(
                    table_hbm.at[idx_v.at[i]],
                    rows_v.at[b],
                    gsem.at[b],
                ).start()
            for b in range(_NBUF):
                i = g * _NBUF + b
                start = base + i * _CHUNK
                pltpu.make_async_copy(
                    table_hbm.at[idx_v.at[i]],
                    rows_v.at[b],
                    gsem.at[b],
                ).wait()
                pltpu.make_async_copy(
                    rows_v.at[b, slice(None), pl.ds(0, EMB_DIM)],
                    out_hbm.at[pl.ds(start, _CHUNK), pl.ds(0, EMB_DIM)],
                    ssem.at[b],
                ).start()
            return carry

        lax.fori_loop(0, _N_GROUPS, group, 0)
        for b in range(_NBUF):
            pltpu.make_async_copy(
                rows_v.at[b, slice(None), pl.ds(0, EMB_DIM)],
                out_hbm.at[pl.ds(base, _CHUNK), pl.ds(0, EMB_DIM)],
                ssem.at[b],
            ).wait()

    return k(x_2d, table_p)


def kernel(x, embedding_weight):
    x_2d = x.reshape(_N // _CHUNK, _CHUNK).astype(jnp.int32)
    table_p = jnp.pad(embedding_weight, ((0, 0), (0, PAD_DIM - EMB_DIM)))
    out_p = _gather_sc(x_2d, table_p)
    return out_p[:, :EMB_DIM].reshape(BATCH, HIST_LEN, EMB_DIM)


# speedup vs baseline: 1.4868x; 1.1767x over previous
import functools

import jax
import jax.numpy as jnp
from jax import lax
from jax.experimental import pallas as pl
from jax.experimental.pallas import tpu as pltpu
from jax.experimental.pallas import tpu_sc as plsc

N_TOKENS = 1000002
EMB_DIM = 64
PAD_DIM = 128
BATCH = 4096
HIST_LEN = 200

_INFO = plsc.get_sparse_core_info()
_NC = _INFO.num_cores
_NS = _INFO.num_subcores
_NW = _NC * _NS

_N = BATCH * HIST_LEN
_B_PER_W = _N // _NW
_CHUNK = 128
_NBUF = 4
_N_CHUNKS = _B_PER_W // _CHUNK
_N_GROUPS = _N_CHUNKS // _NBUF


@jax.jit
def _gather_sc(x_2d, table_p):
    mesh = plsc.VectorSubcoreMesh(core_axis_name="c", subcore_axis_name="s")

    @functools.partial(
        pl.kernel,
        mesh=mesh,
        out_type=jax.ShapeDtypeStruct((_N, PAD_DIM), jnp.float32),
        scratch_types=[
            pltpu.VMEM((_N_CHUNKS, _CHUNK), jnp.int32),
            pltpu.VMEM((_NBUF, _CHUNK, EMB_DIM), jnp.float32),
            pltpu.SemaphoreType.DMA((_NBUF,)),
            pltpu.SemaphoreType.DMA((_NBUF,)),
        ],
        compiler_params=pltpu.CompilerParams(use_tc_tiling_on_sc=False),
    )
    def k(x_hbm, table_hbm, out_hbm, idx_v, rows_v, gsem, ssem):
        wid = lax.axis_index("s") * _NC + lax.axis_index("c")
        base = wid * _B_PER_W
        pltpu.sync_copy(
            x_hbm.at[pl.ds(wid * _N_CHUNKS, _N_CHUNKS), :], idx_v
        )

        def group(g, carry):
            for b in range(_NBUF):
                i = g * _NBUF + b

                @pl.when(g > 0)
                def _wait_store():
                    pltpu.make_async_copy(
                        rows_v.at[b],
                        out_hbm.at[pl.ds(base, _CHUNK), pl.ds(0, EMB_DIM)],
                        ssem.at[b],
                    ).wait()

                pltpu.make_async_copy(
                    table_hbm.at[idx_v.at[i]],
                    rows_v.at[b],
                    gsem.at[b],
                ).start()
            for b in range(_NBUF):
                i = g * _NBUF + b
                start = base + i * _CHUNK
                pltpu.make_async_copy(
                    table_hbm.at[idx_v.at[i]],
                    rows_v.at[b],
                    gsem.at[b],
                ).wait()
                pltpu.make_async_copy(
                    rows_v.at[b],
                    out_hbm.at[pl.ds(start, _CHUNK), pl.ds(0, EMB_DIM)],
                    ssem.at[b],
                ).start()
            return carry

        lax.fori_loop(0, _N_GROUPS, group, 0)
        for b in range(_NBUF):
            pltpu.make_async_copy(
                rows_v.at[b],
                out_hbm.at[pl.ds(base, _CHUNK), pl.ds(0, EMB_DIM)],
                ssem.at[b],
            ).wait()

    return k(x_2d, table_p)


def kernel(x, embedding_weight):
    x_2d = (x.astype(jnp.int32) * 2).reshape(_N // _CHUNK, _CHUNK)
    table_p = jnp.pad(
        embedding_weight, ((0, 0), (0, PAD_DIM - EMB_DIM))
    ).reshape(2 * N_TOKENS, EMB_DIM)
    out_p = _gather_sc(x_2d, table_p)
    return out_p[:, :EMB_DIM].reshape(BATCH, HIST_LEN, EMB_DIM)
